# baseline (speedup 1.0000x reference)
import functools

import jax
import jax.numpy as jnp
from jax import lax
from jax.experimental import pallas as pl
from jax.experimental.pallas import tpu as pltpu
from jax.experimental.pallas import tpu_sc as plsc

B = 8192
D = 1024
H = 512
E = 8
NE = E - 1
BT = 256
NBLK = B // BT + 8
NPAD = NBLK * BT

NC = 2
NS = 16
NW = NC * NS
CHUNK = B // NW
LANES = 16

_SC_MESH = plsc.VectorSubcoreMesh(core_axis_name="c", subcore_axis_name="s")


def _wid():
    return lax.axis_index("s") * NC + lax.axis_index("c")


def _splat_sum(v):
    incl = plsc.cumsum(v)
    rincl = lax.rev(plsc.cumsum(lax.rev(v, (0,))), (0,))
    return incl + rincl - v


def _dot(a, b):
    return lax.dot_general(a, b, (((1,), (0,)), ((), ())),
                           preferred_element_type=jnp.float32)


RBT = 512


def _router_body(x_ref, wr1_ref, br1_ref, wr2_ref, br2_ref,
                 ws_ref, wt_ref, idx_ref):
    xb = x_ref[...]
    r = _dot(xb, wr1_ref[...]) + br1_ref[...]
    r = 0.5 * r * (1.0 + lax.erf(r * 0.7071067811865476))
    logits = _dot(r, wr2_ref[...]) + br2_ref[...]
    m = jnp.max(logits, axis=1, keepdims=True)
    p = jnp.exp(logits - m)
    w = p / jnp.sum(p, axis=1, keepdims=True)
    shared_w = w[:, 0:1]
    col = lax.broadcasted_iota(jnp.int32, (RBT, E), 1)
    wother = jnp.where(col >= 1, w, -1.0)
    top1 = jnp.max(wother, axis=1, keepdims=True)
    is_max = (wother == top1) & (col >= 1)
    eidx = jnp.min(jnp.where(is_max, col, 127), axis=1, keepdims=True) - 1
    denom = shared_w + top1 + 1e-8
    ws_ref[...] = (shared_w / denom)[:, 0]
    wt_ref[...] = (top1 / denom)[:, 0]
    idx_ref[...] = eidx[:, 0].astype(jnp.int32)


def _router(x, Wr1, br1, Wr2, br2):
    nb = B // RBT
    return pl.pallas_call(
        _router_body,
        grid=(nb,),
        in_specs=[
            pl.BlockSpec((RBT, D), lambda b: (b, 0)),
            pl.BlockSpec((D, H), lambda b: (0, 0)),
            pl.BlockSpec((1, H), lambda b: (0, 0)),
            pl.BlockSpec((H, E), lambda b: (0, 0)),
            pl.BlockSpec((1, E), lambda b: (0, 0)),
        ],
        out_specs=[
            pl.BlockSpec((RBT,), lambda b: (b,)),
            pl.BlockSpec((RBT,), lambda b: (b,)),
            pl.BlockSpec((RBT,), lambda b: (b,)),
        ],
        out_shape=[
            jax.ShapeDtypeStruct((B,), jnp.float32),
            jax.ShapeDtypeStruct((B,), jnp.float32),
            jax.ShapeDtypeStruct((B,), jnp.int32),
        ],
    )(x, Wr1, br1.reshape(1, H), Wr2, br2.reshape(1, E))


@functools.partial(
    pl.kernel, mesh=_SC_MESH,
    compiler_params=pltpu.CompilerParams(needs_layout_passes=False),
    out_type=jax.ShapeDtypeStruct((NW, LANES), jnp.int32),
    scratch_types=[pltpu.VMEM((CHUNK,), jnp.int32),
                   pltpu.VMEM((LANES,), jnp.int32)],
)
def _sc_hist(idx_hbm, hist_hbm, idx_v, hist_v):
    wid = _wid()
    base = wid * CHUNK
    pltpu.sync_copy(idx_hbm.at[pl.ds(base, CHUNK)], idx_v)
    lane = lax.iota(jnp.int32, LANES)
    counts = jnp.zeros((LANES,), jnp.int32)
    for v in range(CHUNK // LANES):
        vec = idx_v[pl.ds(v * LANES, LANES)]
        for e in range(NE):
            ind = (vec == e).astype(jnp.int32)
            counts = counts + jnp.where(lane == e, _splat_sum(ind), 0)
    hist_v[...] = counts
    pltpu.sync_copy(hist_v, hist_hbm.at[wid])


@functools.partial(
    pl.kernel, mesh=_SC_MESH,
    compiler_params=pltpu.CompilerParams(needs_layout_passes=False),
    out_type=[jax.ShapeDtypeStruct((B,), jnp.int32),
              jax.ShapeDtypeStruct((2, 48), jnp.int32)],
    scratch_types=[pltpu.VMEM((CHUNK,), jnp.int32),
                   pltpu.VMEM((NW, LANES), jnp.int32),
                   pltpu.VMEM((LANES,), jnp.int32),
                   pltpu.VMEM((CHUNK,), jnp.int32),
                   pltpu.VMEM((2, 48), jnp.int32)],
)
def _sc_pos(idx_hbm, hist_hbm, pos_hbm, gid_hbm,
            idx_v, hist_v, base_v, pos_v, gid_v):
    wid = _wid()
    base = wid * CHUNK
    lane = lax.iota(jnp.int32, LANES)
    pltpu.sync_copy(hist_hbm, hist_v)
    pltpu.sync_copy(idx_hbm.at[pl.ds(base, CHUNK)], idx_v)

    total = jnp.zeros((LANES,), jnp.int32)
    prefix = jnp.zeros((LANES,), jnp.int32)
    for t in range(NW):
        row = hist_v[t]
        total = total + row
        before = jnp.full((LANES,), t, jnp.int32) < wid
        prefix = prefix + jnp.where(before, row, 0)
    padded = ((total + (BT - 1)) // BT) * BT
    offs = plsc.cumsum(padded) - padded
    base_v[...] = offs + prefix

    counter = jnp.zeros((LANES,), jnp.int32)
    for v in range(CHUNK // LANES):
        vec = idx_v[pl.ds(v * LANES, LANES)]
        rank = jnp.zeros((LANES,), jnp.int32)
        for e in range(NE):
            m = vec == e
            ind = m.astype(jnp.int32)
            excl = plsc.cumsum(ind) - ind
            ce = _splat_sum(jnp.where(lane == e, counter, 0))
            rank = rank + jnp.where(m, excl + ce, 0)
            counter = counter + jnp.where(lane == e, _splat_sum(ind), 0)
        seg = plsc.load_gather(base_v, [vec])
        pos_v[pl.ds(v * LANES, LANES)] = seg + rank
    pltpu.sync_copy(pos_v, pos_hbm.at[pl.ds(base, CHUNK)])

    @pl.when(wid == 0)
    def _gid():
        tp = _splat_sum(padded)
        for v in range(48 // LANES):
            bstart = (lane + v * LANES) * BT
            cnt = jnp.zeros((LANES,), jnp.int32)
            for e in range(NE):
                off_e = _splat_sum(jnp.where(lane == e, offs, 0))
                cnt = cnt + (bstart >= off_e).astype(jnp.int32)
            gid_v[0, pl.ds(v * LANES, LANES)] = jnp.clip(cnt - 1, 0, NE - 1)
            gid_v[1, pl.ds(v * LANES, LANES)] = (bstart < tp).astype(jnp.int32)
        pltpu.sync_copy(gid_v, gid_hbm)


XCH = 32
NCH = CHUNK // XCH


@functools.partial(
    pl.kernel, mesh=_SC_MESH,
    compiler_params=pltpu.CompilerParams(needs_layout_passes=False),
    out_type=[jax.ShapeDtypeStruct((NPAD, D), jnp.float32),
              jax.ShapeDtypeStruct((NPAD, 128), jnp.float32)],
    scratch_types=[pltpu.VMEM((CHUNK,), jnp.int32)]
                  + [pltpu.VMEM((XCH,), jnp.int32)] * NCH
                  + [pltpu.VMEM((XCH, D), jnp.float32),
                     pltpu.VMEM((XCH, D), jnp.float32),
                     pltpu.VMEM((CHUNK, 128), jnp.float32),
                     pltpu.VMEM((CHUNK,), jnp.float32),
                     pltpu.VMEM((CHUNK,), jnp.float32)]
                  + [pltpu.SemaphoreType.DMA] * 5,
)
def _sc_dispatch(x_hbm, pos_hbm, wt_hbm, ws_hbm, xs_hbm, wtws_hbm,
                 pos_v, p0, p1, p2, p3, p4, p5, p6, p7, rows0, rows1,
                 wtws_v, wt_v, ws_v, semr0, semr1, semw0, semw1, semw2):
    wid = _wid()
    base = wid * CHUNK
    lane = lax.iota(jnp.int32, LANES)
    pcs = (p0, p1, p2, p3, p4, p5, p6, p7)
    rows = (rows0, rows1)
    semr = (semr0, semr1)
    semw = (semw0, semw1)
    pltpu.sync_copy(pos_hbm.at[pl.ds(base, CHUNK)], pos_v)
    pltpu.sync_copy(wt_hbm.at[pl.ds(base, CHUNK)], wt_v)
    pltpu.sync_copy(ws_hbm.at[pl.ds(base, CHUNK)], ws_v)
    for c in range(NCH):
        pltpu.sync_copy(pos_hbm.at[pl.ds(base + c * XCH, XCH)], pcs[c])
    zero = jnp.zeros((LANES,), jnp.int32)
    for v in range(CHUNK // LANES):
        row = lane + v * LANES
        plsc.store_scatter(wtws_v, [row, zero], wt_v[pl.ds(v * LANES, LANES)])
        plsc.store_scatter(wtws_v, [row, zero + 1],
                           ws_v[pl.ds(v * LANES, LANES)])
    wtcopy = pltpu.async_copy(wtws_v, wtws_hbm.at[pos_v], semw2)
    rd = [None] * NCH
    wr = [None] * NCH
    rd[0] = pltpu.async_copy(x_hbm.at[pl.ds(base, XCH)], rows[0], semr[0])
    for c in range(NCH):
        if c >= 1:
            wr[c - 1].wait()
        if c + 1 < NCH:
            rd[c + 1] = pltpu.async_copy(
                x_hbm.at[pl.ds(base + (c + 1) * XCH, XCH)],
                rows[(c + 1) % 2], semr[(c + 1) % 2])
        rd[c].wait()
        wr[c] = pltpu.async_copy(rows[c % 2], xs_hbm.at[pcs[c]], semw[c % 2])
    wr[NCH - 1].wait()
    wtcopy.wait()


def _moe_body(gid_ref, xs_ref, wtws_ref,
              sw1_ref, sb1_ref, sw2_ref, sb2_ref, sw3_ref, sb3_ref,
              ew1_ref, eb1_ref, ew2_ref, eb2_ref, ew3_ref, eb3_ref,
              m1w_ref, m1b_ref, m2w_ref, m2b_ref, m3w_ref, m3b_ref,
              out_ref):
    b = pl.program_id(0)

    @pl.when(gid_ref[1, b] == 1)
    def _valid_block():
        _moe_block(xs_ref, wtws_ref,
                   sw1_ref, sb1_ref, sw2_ref, sb2_ref, sw3_ref, sb3_ref,
                   ew1_ref, eb1_ref, ew2_ref, eb2_ref, ew3_ref, eb3_ref,
                   m1w_ref, m1b_ref, m2w_ref, m2b_ref, m3w_ref, m3b_ref,
                   out_ref)


def _moe_block(xs_ref, wtws_ref,
               sw1_ref, sb1_ref, sw2_ref, sb2_ref, sw3_ref, sb3_ref,
               ew1_ref, eb1_ref, ew2_ref, eb2_ref, ew3_ref, eb3_ref,
               m1w_ref, m1b_ref, m2w_ref, m2b_ref, m3w_ref, m3b_ref,
               out_ref):
    xb = xs_ref[...]
    s = jax.nn.silu(_dot(xb, sw1_ref[...]) + sb1_ref[...])
    s = s + _dot(xb, sw3_ref[...]) + sb3_ref[...]
    so = _dot(s, sw2_ref[...]) + sb2_ref[...]
    h = jax.nn.silu(_dot(xb, ew1_ref[0]) + eb1_ref[0])
    h = h + _dot(xb, ew3_ref[0]) + eb3_ref[0]
    eo = _dot(h, ew2_ref[0]) + eb2_ref[0]
    wt = wtws_ref[:, 0:1]
    ws = wtws_ref[:, 1:2]
    c = ws * so + wt * eo + xb
    y = jnp.maximum(_dot(c, m1w_ref[...]) + m1b_ref[...], 0.0)
    y = jnp.maximum(_dot(y, m2w_ref[...]) + m2b_ref[...], 0.0)
    out_ref[...] = jnp.tanh(_dot(y, m3w_ref[...]) + m3b_ref[...])


def _moe(gid, xs, wtws, Sw1, Sb1, Sw2, Sb2, Sw3, Sb3,
         Ew1, Eb1, Ew2, Eb2, Ew3, Eb3, M1w, M1b, M2w, M2b, M3w, M3b):
    full2 = lambda shape: pl.BlockSpec(shape, lambda b, g: (0, 0))
    grid_spec = pltpu.PrefetchScalarGridSpec(
        num_scalar_prefetch=1,
        grid=(NBLK,),
        in_specs=[
            pl.BlockSpec((BT, D), lambda b, g: (b, 0)),
            pl.BlockSpec((BT, 128), lambda b, g: (b, 0)),
            full2((D, H)), full2((1, H)),
            full2((H, D)), full2((1, D)),
            full2((D, H)), full2((1, H)),
            pl.BlockSpec((1, D, H), lambda b, g: (g[0, b], 0, 0)),
            pl.BlockSpec((1, 1, H), lambda b, g: (g[0, b], 0, 0)),
            pl.BlockSpec((1, H, D), lambda b, g: (g[0, b], 0, 0)),
            pl.BlockSpec((1, 1, D), lambda b, g: (g[0, b], 0, 0)),
            pl.BlockSpec((1, D, H), lambda b, g: (g[0, b], 0, 0)),
            pl.BlockSpec((1, 1, H), lambda b, g: (g[0, b], 0, 0)),
            full2((D, 256)), full2((1, 256)),
            full2((256, 256)), full2((1, 256)),
            full2((256, LANES)), full2((1, LANES)),
        ],
        out_specs=pl.BlockSpec((BT, LANES), lambda b, g: (b, 0)),
    )
    m3wp = jnp.pad(M3w, ((0, 0), (0, LANES - 4)))
    m3bp = jnp.pad(M3b.reshape(1, 4), ((0, 0), (0, LANES - 4)))
    return pl.pallas_call(
        _moe_body,
        grid_spec=grid_spec,
        out_shape=jax.ShapeDtypeStruct((NPAD, LANES), jnp.float32),
    )(gid, xs, wtws, Sw1, Sb1.reshape(1, H), Sw2, Sb2.reshape(1, D),
      Sw3, Sb3.reshape(1, H),
      Ew1, Eb1.reshape(NE, 1, H), Ew2, Eb2.reshape(NE, 1, D),
      Ew3, Eb3.reshape(NE, 1, H),
      M1w, M1b.reshape(1, 256), M2w, M2b.reshape(1, 256), m3wp, m3bp)


@functools.partial(
    pl.kernel, mesh=_SC_MESH,
    compiler_params=pltpu.CompilerParams(needs_layout_passes=False,
                                         use_tc_tiling_on_sc=False),
    out_type=jax.ShapeDtypeStruct((B, 4), jnp.float32),
    scratch_types=[pltpu.VMEM((CHUNK,), jnp.int32),
                   pltpu.VMEM((CHUNK, LANES), jnp.float32),
                   pltpu.SemaphoreType.DMA],
)
def _sc_unperm(head_hbm, pos_hbm, out_hbm, pos_v, rows_v, sem):
    wid = _wid()
    base = wid * CHUNK
    pltpu.sync_copy(pos_hbm.at[pl.ds(base, CHUNK)], pos_v)
    pltpu.async_copy(head_hbm.at[pos_v], rows_v, sem).wait()
    pltpu.sync_copy(rows_v.at[:, pl.ds(0, 4)], out_hbm.at[pl.ds(base, CHUNK)])


def kernel(x, Wr1, br1, Wr2, br2, Sw1, Sb1, Sw2, Sb2, Sw3, Sb3,
           Ew1, Eb1, Ew2, Eb2, Ew3, Eb3, M1w, M1b, M2w, M2b, M3w, M3b):
    ws, wt, eidx = _router(x, Wr1, br1, Wr2, br2)
    hist = _sc_hist(eidx)
    pos, gid = _sc_pos(eidx, hist)
    xs, wtws = _sc_dispatch(x, pos, wt, ws)
    head = _moe(gid, xs, wtws, Sw1, Sb1, Sw2, Sb2, Sw3, Sb3,
                Ew1, Eb1, Ew2, Eb2, Ew3, Eb3, M1w, M1b, M2w, M2b, M3w, M3b)
    return _sc_unperm(head, pos)


# speedup vs baseline: 1.0327x; 1.0327x over previous
import functools

import jax
import jax.numpy as jnp
from jax import lax
from jax.experimental import pallas as pl
from jax.experimental.pallas import tpu as pltpu
from jax.experimental.pallas import tpu_sc as plsc

B = 8192
NH = 2
HB = B // NH
D = 1024
H = 512
E = 8
NE = E - 1
BT = 256
NBLK = HB // BT + 8
NPAD = NBLK * BT
NGID = 48

NC = 2
NS = 16
NW = NC * NS
CHUNK = HB // NW
LANES = 16
XCH = 32
NCH = CHUNK // XCH

_SC_MESH = plsc.VectorSubcoreMesh(core_axis_name="c", subcore_axis_name="s")


def _wid():
    return lax.axis_index("s") * NC + lax.axis_index("c")


def _splat_sum(v):
    incl = plsc.cumsum(v)
    rincl = lax.rev(plsc.cumsum(lax.rev(v, (0,))), (0,))
    return incl + rincl - v


def _dot(a, b):
    return lax.dot_general(a, b, (((1,), (0,)), ((), ())),
                           preferred_element_type=jnp.float32)


RBT = 512


def _router_body(x_ref, wr1_ref, br1_ref, wr2_ref, br2_ref,
                 ws_ref, wt_ref, idx_ref):
    xb = x_ref[...]
    r = _dot(xb, wr1_ref[...]) + br1_ref[...]
    r = 0.5 * r * (1.0 + lax.erf(r * 0.7071067811865476))
    logits = _dot(r, wr2_ref[...]) + br2_ref[...]
    m = jnp.max(logits, axis=1, keepdims=True)
    p = jnp.exp(logits - m)
    w = p / jnp.sum(p, axis=1, keepdims=True)
    shared_w = w[:, 0:1]
    col = lax.broadcasted_iota(jnp.int32, (RBT, E), 1)
    wother = jnp.where(col >= 1, w, -1.0)
    top1 = jnp.max(wother, axis=1, keepdims=True)
    is_max = (wother == top1) & (col >= 1)
    eidx = jnp.min(jnp.where(is_max, col, 127), axis=1, keepdims=True) - 1
    denom = shared_w + top1 + 1e-8
    ws_ref[...] = shared_w / denom
    wt_ref[...] = top1 / denom
    idx_ref[...] = eidx.astype(jnp.int32)


def _make_router(h):
    nb = HB // RBT

    def run(x, Wr1, br1, Wr2, br2):
        return pl.pallas_call(
            _router_body,
            grid=(nb,),
            in_specs=[
                pl.BlockSpec((RBT, D), lambda b: (b + h * nb, 0)),
                pl.BlockSpec((D, H), lambda b: (0, 0)),
                pl.BlockSpec((1, H), lambda b: (0, 0)),
                pl.BlockSpec((H, E), lambda b: (0, 0)),
                pl.BlockSpec((1, E), lambda b: (0, 0)),
            ],
            out_specs=[
                pl.BlockSpec((RBT, 1), lambda b: (b, 0)),
                pl.BlockSpec((RBT, 1), lambda b: (b, 0)),
                pl.BlockSpec((RBT, 1), lambda b: (b, 0)),
            ],
            out_shape=[
                jax.ShapeDtypeStruct((HB, 1), jnp.float32),
                jax.ShapeDtypeStruct((HB, 1), jnp.float32),
                jax.ShapeDtypeStruct((HB, 1), jnp.int32),
            ],
        )(x, Wr1, br1.reshape(1, H), Wr2, br2.reshape(1, E))

    return run


@functools.partial(
    pl.kernel, mesh=_SC_MESH,
    compiler_params=pltpu.CompilerParams(needs_layout_passes=False),
    out_type=jax.ShapeDtypeStruct((NW, LANES), jnp.int32),
    scratch_types=[pltpu.VMEM((CHUNK,), jnp.int32),
                   pltpu.VMEM((LANES,), jnp.int32)],
)
def _sc_hist(idx_hbm, hist_hbm, idx_v, hist_v):
    wid = _wid()
    base = wid * CHUNK
    pltpu.sync_copy(idx_hbm.at[pl.ds(base, CHUNK)], idx_v)
    lane = lax.iota(jnp.int32, LANES)
    counts = jnp.zeros((LANES,), jnp.int32)
    for v in range(CHUNK // LANES):
        vec = idx_v[pl.ds(v * LANES, LANES)]
        for e in range(NE):
            ind = (vec == e).astype(jnp.int32)
            counts = counts + jnp.where(lane == e, _splat_sum(ind), 0)
    hist_v[...] = counts
    pltpu.sync_copy(hist_v, hist_hbm.at[wid])


@functools.partial(
    pl.kernel, mesh=_SC_MESH,
    compiler_params=pltpu.CompilerParams(needs_layout_passes=False),
    out_type=[jax.ShapeDtypeStruct((HB,), jnp.int32),
              jax.ShapeDtypeStruct((2, NGID), jnp.int32)],
    scratch_types=[pltpu.VMEM((CHUNK,), jnp.int32),
                   pltpu.VMEM((NW, LANES), jnp.int32),
                   pltpu.VMEM((LANES,), jnp.int32),
                   pltpu.VMEM((CHUNK,), jnp.int32),
                   pltpu.VMEM((2, NGID), jnp.int32)],
)
def _sc_pos(idx_hbm, hist_hbm, pos_hbm, gid_hbm,
            idx_v, hist_v, base_v, pos_v, gid_v):
    wid = _wid()
    base = wid * CHUNK
    lane = lax.iota(jnp.int32, LANES)
    pltpu.sync_copy(hist_hbm, hist_v)
    pltpu.sync_copy(idx_hbm.at[pl.ds(base, CHUNK)], idx_v)

    total = jnp.zeros((LANES,), jnp.int32)
    prefix = jnp.zeros((LANES,), jnp.int32)
    for t in range(NW):
        row = hist_v[t]
        total = total + row
        before = jnp.full((LANES,), t, jnp.int32) < wid
        prefix = prefix + jnp.where(before, row, 0)
    padded = ((total + (BT - 1)) // BT) * BT
    offs = plsc.cumsum(padded) - padded
    base_v[...] = offs + prefix

    counter = jnp.zeros((LANES,), jnp.int32)
    for v in range(CHUNK // LANES):
        vec = idx_v[pl.ds(v * LANES, LANES)]
        rank = jnp.zeros((LANES,), jnp.int32)
        for e in range(NE):
            m = vec == e
            ind = m.astype(jnp.int32)
            excl = plsc.cumsum(ind) - ind
            ce = _splat_sum(jnp.where(lane == e, counter, 0))
            rank = rank + jnp.where(m, excl + ce, 0)
            counter = counter + jnp.where(lane == e, _splat_sum(ind), 0)
        seg = plsc.load_gather(base_v, [vec])
        pos_v[pl.ds(v * LANES, LANES)] = seg + rank
    pltpu.sync_copy(pos_v, pos_hbm.at[pl.ds(base, CHUNK)])

    @pl.when(wid == 0)
    def _gid():
        tp = _splat_sum(padded)
        for v in range(NGID // LANES):
            bstart = (lane + v * LANES) * BT
            cnt = jnp.zeros((LANES,), jnp.int32)
            for e in range(NE):
                off_e = _splat_sum(jnp.where(lane == e, offs, 0))
                cnt = cnt + (bstart >= off_e).astype(jnp.int32)
            gid_v[0, pl.ds(v * LANES, LANES)] = jnp.clip(cnt - 1, 0, NE - 1)
            gid_v[1, pl.ds(v * LANES, LANES)] = (bstart < tp).astype(jnp.int32)
        pltpu.sync_copy(gid_v, gid_hbm)


def _make_dispatch(h):
    off = h * HB

    @functools.partial(
        pl.kernel, mesh=_SC_MESH,
        compiler_params=pltpu.CompilerParams(needs_layout_passes=False),
        out_type=[jax.ShapeDtypeStruct((NPAD, D), jnp.float32),
                  jax.ShapeDtypeStruct((NPAD, 128), jnp.float32)],
        scratch_types=[pltpu.VMEM((CHUNK,), jnp.int32)]
                      + [pltpu.VMEM((XCH,), jnp.int32)] * NCH
                      + [pltpu.VMEM((XCH, D), jnp.float32),
                         pltpu.VMEM((XCH, D), jnp.float32),
                         pltpu.VMEM((CHUNK, 128), jnp.float32),
                         pltpu.VMEM((CHUNK,), jnp.float32),
                         pltpu.VMEM((CHUNK,), jnp.float32)]
                      + [pltpu.SemaphoreType.DMA] * 5,
    )
    def _sc_dispatch(x_hbm, pos_hbm, wt_hbm, ws_hbm, xs_hbm, wtws_hbm,
                     pos_v, p0, p1, p2, p3, rows0, rows1,
                     wtws_v, wt_v, ws_v, semr0, semr1, semw0, semw1, semw2):
        wid = _wid()
        base = wid * CHUNK
        lane = lax.iota(jnp.int32, LANES)
        pcs = (p0, p1, p2, p3)
        rows = (rows0, rows1)
        semr = (semr0, semr1)
        semw = (semw0, semw1)
        pltpu.sync_copy(pos_hbm.at[pl.ds(base, CHUNK)], pos_v)
        pltpu.sync_copy(wt_hbm.at[pl.ds(base, CHUNK)], wt_v)
        pltpu.sync_copy(ws_hbm.at[pl.ds(base, CHUNK)], ws_v)
        for c in range(NCH):
            pltpu.sync_copy(pos_hbm.at[pl.ds(base + c * XCH, XCH)], pcs[c])
        zero = jnp.zeros((LANES,), jnp.int32)
        for v in range(CHUNK // LANES):
            row = lane + v * LANES
            plsc.store_scatter(wtws_v, [row, zero],
                               wt_v[pl.ds(v * LANES, LANES)])
            plsc.store_scatter(wtws_v, [row, zero + 1],
                               ws_v[pl.ds(v * LANES, LANES)])
        wtcopy = pltpu.async_copy(wtws_v, wtws_hbm.at[pos_v], semw2)
        rd = [None] * NCH
        wr = [None] * NCH
        rd[0] = pltpu.async_copy(x_hbm.at[pl.ds(off + base, XCH)],
                                 rows[0], semr[0])
        for c in range(NCH):
            if c >= 1:
                wr[c - 1].wait()
            if c + 1 < NCH:
                rd[c + 1] = pltpu.async_copy(
                    x_hbm.at[pl.ds(off + base + (c + 1) * XCH, XCH)],
                    rows[(c + 1) % 2], semr[(c + 1) % 2])
            rd[c].wait()
            wr[c] = pltpu.async_copy(rows[c % 2], xs_hbm.at[pcs[c]],
                                     semw[c % 2])
        wr[NCH - 1].wait()
        wtcopy.wait()

    return _sc_dispatch


def _moe_body(gid_ref, xs_ref, wtws_ref,
              sw1_ref, sb1_ref, sw2_ref, sb2_ref, sw3_ref, sb3_ref,
              ew1_ref, eb1_ref, ew2_ref, eb2_ref, ew3_ref, eb3_ref,
              m1w_ref, m1b_ref, m2w_ref, m2b_ref, m3w_ref, m3b_ref,
              out_ref):
    b = pl.program_id(0)

    @pl.when(gid_ref[1, b] == 1)
    def _valid_block():
        _moe_block(xs_ref, wtws_ref,
                   sw1_ref, sb1_ref, sw2_ref, sb2_ref, sw3_ref, sb3_ref,
                   ew1_ref, eb1_ref, ew2_ref, eb2_ref, ew3_ref, eb3_ref,
                   m1w_ref, m1b_ref, m2w_ref, m2b_ref, m3w_ref, m3b_ref,
                   out_ref)


def _moe_block(xs_ref, wtws_ref,
               sw1_ref, sb1_ref, sw2_ref, sb2_ref, sw3_ref, sb3_ref,
               ew1_ref, eb1_ref, ew2_ref, eb2_ref, ew3_ref, eb3_ref,
               m1w_ref, m1b_ref, m2w_ref, m2b_ref, m3w_ref, m3b_ref,
               out_ref):
    xb = xs_ref[...]
    s = jax.nn.silu(_dot(xb, sw1_ref[...]) + sb1_ref[...])
    s = s + _dot(xb, sw3_ref[...]) + sb3_ref[...]
    so = _dot(s, sw2_ref[...]) + sb2_ref[...]
    h = jax.nn.silu(_dot(xb, ew1_ref[0]) + eb1_ref[0])
    h = h + _dot(xb, ew3_ref[0]) + eb3_ref[0]
    eo = _dot(h, ew2_ref[0]) + eb2_ref[0]
    wt = wtws_ref[:, 0:1]
    ws = wtws_ref[:, 1:2]
    c = ws * so + wt * eo + xb
    y = jnp.maximum(_dot(c, m1w_ref[...]) + m1b_ref[...], 0.0)
    y = jnp.maximum(_dot(y, m2w_ref[...]) + m2b_ref[...], 0.0)
    out_ref[...] = jnp.tanh(_dot(y, m3w_ref[...]) + m3b_ref[...])


def _moe(gid, xs, wtws, Sw1, Sb1, Sw2, Sb2, Sw3, Sb3,
         Ew1, Eb1, Ew2, Eb2, Ew3, Eb3, M1w, M1b, M2w, M2b, M3w, M3b):
    full2 = lambda shape: pl.BlockSpec(shape, lambda b, g: (0, 0))
    grid_spec = pltpu.PrefetchScalarGridSpec(
        num_scalar_prefetch=1,
        grid=(NBLK,),
        in_specs=[
            pl.BlockSpec((BT, D), lambda b, g: (b, 0)),
            pl.BlockSpec((BT, 128), lambda b, g: (b, 0)),
            full2((D, H)), full2((1, H)),
            full2((H, D)), full2((1, D)),
            full2((D, H)), full2((1, H)),
            pl.BlockSpec((1, D, H), lambda b, g: (g[0, b], 0, 0)),
            pl.BlockSpec((1, 1, H), lambda b, g: (g[0, b], 0, 0)),
            pl.BlockSpec((1, H, D), lambda b, g: (g[0, b], 0, 0)),
            pl.BlockSpec((1, 1, D), lambda b, g: (g[0, b], 0, 0)),
            pl.BlockSpec((1, D, H), lambda b, g: (g[0, b], 0, 0)),
            pl.BlockSpec((1, 1, H), lambda b, g: (g[0, b], 0, 0)),
            full2((D, 256)), full2((1, 256)),
            full2((256, 256)), full2((1, 256)),
            full2((256, LANES)), full2((1, LANES)),
        ],
        out_specs=pl.BlockSpec((BT, LANES), lambda b, g: (b, 0)),
    )
    m3wp = jnp.pad(M3w, ((0, 0), (0, LANES - 4)))
    m3bp = jnp.pad(M3b.reshape(1, 4), ((0, 0), (0, LANES - 4)))
    return pl.pallas_call(
        _moe_body,
        grid_spec=grid_spec,
        out_shape=jax.ShapeDtypeStruct((NPAD, LANES), jnp.float32),
    )(gid, xs, wtws, Sw1, Sb1.reshape(1, H), Sw2, Sb2.reshape(1, D),
      Sw3, Sb3.reshape(1, H),
      Ew1, Eb1.reshape(NE, 1, H), Ew2, Eb2.reshape(NE, 1, D),
      Ew3, Eb3.reshape(NE, 1, H),
      M1w, M1b.reshape(1, 256), M2w, M2b.reshape(1, 256), m3wp, m3bp)


@functools.partial(
    pl.kernel, mesh=_SC_MESH,
    compiler_params=pltpu.CompilerParams(needs_layout_passes=False,
                                         use_tc_tiling_on_sc=False),
    out_type=jax.ShapeDtypeStruct((HB, LANES), jnp.float32),
    scratch_types=[pltpu.VMEM((CHUNK,), jnp.int32),
                   pltpu.VMEM((CHUNK, LANES), jnp.float32),
                   pltpu.SemaphoreType.DMA],
)
def _sc_unperm(head_hbm, pos_hbm, out_hbm, pos_v, rows_v, sem):
    wid = _wid()
    base = wid * CHUNK
    pltpu.sync_copy(pos_hbm.at[pl.ds(base, CHUNK)], pos_v)
    pltpu.async_copy(head_hbm.at[pos_v], rows_v, sem).wait()
    pltpu.sync_copy(rows_v, out_hbm.at[pl.ds(base, CHUNK)])


_ROUTERS = (_make_router(0), _make_router(1))
_DISPATCHES = (_make_dispatch(0), _make_dispatch(1))


def kernel(x, Wr1, br1, Wr2, br2, Sw1, Sb1, Sw2, Sb2, Sw3, Sb3,
           Ew1, Eb1, Ew2, Eb2, Ew3, Eb3, M1w, M1b, M2w, M2b, M3w, M3b):
    outs = []
    halves = []
    for hh in range(NH):
        ws, wt, eidx = _ROUTERS[hh](x, Wr1, br1, Wr2, br2)
        hist = _sc_hist(eidx.reshape(HB))
        pos, gid = _sc_pos(eidx.reshape(HB), hist)
        xs, wtws = _DISPATCHES[hh](x, pos, wt.reshape(HB), ws.reshape(HB))
        halves.append((pos, gid, xs, wtws))
    for hh in range(NH):
        pos, gid, xs, wtws = halves[hh]
        head = _moe(gid, xs, wtws, Sw1, Sb1, Sw2, Sb2, Sw3, Sb3,
                    Ew1, Eb1, Ew2, Eb2, Ew3, Eb3, M1w, M1b, M2w, M2b, M3w,
                    M3b)
        outs.append(_sc_unperm(head, pos))
    return jnp.concatenate(outs, axis=0)[:, :4]


# speedup vs baseline: 1.0926x; 1.0580x over previous
import functools

import jax
import jax.numpy as jnp
from jax import lax
from jax.experimental import pallas as pl
from jax.experimental.pallas import tpu as pltpu
from jax.experimental.pallas import tpu_sc as plsc

B = 8192
D = 1024
H = 512
E = 8
NE = E - 1
BT = 256
NBLK = B // BT + 8
NPAD = NBLK * BT

NC = 2
NS = 16
NW = NC * NS
CHUNK = B // NW
LANES = 16

_SC_MESH = plsc.VectorSubcoreMesh(core_axis_name="c", subcore_axis_name="s")


def _wid():
    return lax.axis_index("s") * NC + lax.axis_index("c")


def _splat_sum(v):
    incl = plsc.cumsum(v)
    rincl = lax.rev(plsc.cumsum(lax.rev(v, (0,))), (0,))
    return incl + rincl - v


def _dot(a, b):
    return lax.dot_general(a, b, (((1,), (0,)), ((), ())),
                           preferred_element_type=jnp.float32)


RBT = 512


def _router_body(x_ref, wr1_ref, br1_ref, wr2_ref, br2_ref,
                 pk_ref, idx_ref):
    xb = x_ref[...]
    r = _dot(xb, wr1_ref[...]) + br1_ref[...]
    r = 0.5 * r * (1.0 + lax.erf(r * 0.7071067811865476))
    logits = _dot(r, wr2_ref[...]) + br2_ref[...]
    m = jnp.max(logits, axis=1, keepdims=True)
    p = jnp.exp(logits - m)
    w = p / jnp.sum(p, axis=1, keepdims=True)
    shared_w = w[:, 0:1]
    col = lax.broadcasted_iota(jnp.int32, (RBT, E), 1)
    wother = jnp.where(col >= 1, w, -1.0)
    top1 = jnp.max(wother, axis=1, keepdims=True)
    is_max = (wother == top1) & (col >= 1)
    eidx = jnp.min(jnp.where(is_max, col, 127), axis=1, keepdims=True) - 1
    denom = shared_w + top1 + 1e-8
    packed = jnp.concatenate(
        [shared_w / denom, top1 / denom,
         jnp.zeros((RBT, 126), jnp.float32)], axis=1)
    pk_ref[...] = packed
    idx_ref[...] = eidx.astype(jnp.int32)


def _router(x, Wr1, br1, Wr2, br2):
    nb = B // RBT
    return pl.pallas_call(
        _router_body,
        grid=(nb,),
        in_specs=[
            pl.BlockSpec((RBT, D), lambda b: (b, 0)),
            pl.BlockSpec((D, H), lambda b: (0, 0)),
            pl.BlockSpec((1, H), lambda b: (0, 0)),
            pl.BlockSpec((H, E), lambda b: (0, 0)),
            pl.BlockSpec((1, E), lambda b: (0, 0)),
        ],
        out_specs=[
            pl.BlockSpec((RBT, 128), lambda b: (b, 0)),
            pl.BlockSpec((RBT, 1), lambda b: (b, 0)),
        ],
        out_shape=[
            jax.ShapeDtypeStruct((B, 128), jnp.float32),
            jax.ShapeDtypeStruct((B, 1), jnp.int32),
        ],
    )(x, Wr1, br1.reshape(1, H), Wr2, br2.reshape(1, E))


@functools.partial(
    pl.kernel, mesh=_SC_MESH,
    compiler_params=pltpu.CompilerParams(needs_layout_passes=False),
    out_type=jax.ShapeDtypeStruct((NW, LANES), jnp.int32),
    scratch_types=[pltpu.VMEM((CHUNK,), jnp.int32),
                   pltpu.VMEM((LANES,), jnp.int32)],
)
def _sc_hist(idx_hbm, hist_hbm, idx_v, hist_v):
    wid = _wid()
    base = wid * CHUNK
    pltpu.sync_copy(idx_hbm.at[pl.ds(base, CHUNK)], idx_v)
    lane = lax.iota(jnp.int32, LANES)
    counts = jnp.zeros((LANES,), jnp.int32)
    for v in range(CHUNK // LANES):
        vec = idx_v[pl.ds(v * LANES, LANES)]
        for e in range(NE):
            ind = (vec == e).astype(jnp.int32)
            counts = counts + jnp.where(lane == e, _splat_sum(ind), 0)
    hist_v[...] = counts
    pltpu.sync_copy(hist_v, hist_hbm.at[wid])


@functools.partial(
    pl.kernel, mesh=_SC_MESH,
    compiler_params=pltpu.CompilerParams(needs_layout_passes=False),
    out_type=[jax.ShapeDtypeStruct((B,), jnp.int32),
              jax.ShapeDtypeStruct((2, 48), jnp.int32)],
    scratch_types=[pltpu.VMEM((CHUNK,), jnp.int32),
                   pltpu.VMEM((NW, LANES), jnp.int32),
                   pltpu.VMEM((LANES,), jnp.int32),
                   pltpu.VMEM((CHUNK,), jnp.int32),
                   pltpu.VMEM((2, 48), jnp.int32)],
)
def _sc_pos(idx_hbm, hist_hbm, pos_hbm, gid_hbm,
            idx_v, hist_v, base_v, pos_v, gid_v):
    wid = _wid()
    base = wid * CHUNK
    lane = lax.iota(jnp.int32, LANES)
    pltpu.sync_copy(hist_hbm, hist_v)
    pltpu.sync_copy(idx_hbm.at[pl.ds(base, CHUNK)], idx_v)

    total = jnp.zeros((LANES,), jnp.int32)
    prefix = jnp.zeros((LANES,), jnp.int32)
    for t in range(NW):
        row = hist_v[t]
        total = total + row
        before = jnp.full((LANES,), t, jnp.int32) < wid
        prefix = prefix + jnp.where(before, row, 0)
    padded = ((total + (BT - 1)) // BT) * BT
    offs = plsc.cumsum(padded) - padded
    base_v[...] = offs + prefix

    counter = jnp.zeros((LANES,), jnp.int32)
    for v in range(CHUNK // LANES):
        vec = idx_v[pl.ds(v * LANES, LANES)]
        rank = jnp.zeros((LANES,), jnp.int32)
        for e in range(NE):
            m = vec == e
            ind = m.astype(jnp.int32)
            excl = plsc.cumsum(ind) - ind
            ce = _splat_sum(jnp.where(lane == e, counter, 0))
            rank = rank + jnp.where(m, excl + ce, 0)
            counter = counter + jnp.where(lane == e, _splat_sum(ind), 0)
        seg = plsc.load_gather(base_v, [vec])
        pos_v[pl.ds(v * LANES, LANES)] = seg + rank
    pltpu.sync_copy(pos_v, pos_hbm.at[pl.ds(base, CHUNK)])

    @pl.when(wid == 0)
    def _gid():
        tp = _splat_sum(padded)
        for v in range(48 // LANES):
            bstart = (lane + v * LANES) * BT
            cnt = jnp.zeros((LANES,), jnp.int32)
            for e in range(NE):
                off_e = _splat_sum(jnp.where(lane == e, offs, 0))
                cnt = cnt + (bstart >= off_e).astype(jnp.int32)
            gid_v[0, pl.ds(v * LANES, LANES)] = jnp.clip(cnt - 1, 0, NE - 1)
            gid_v[1, pl.ds(v * LANES, LANES)] = (bstart < tp).astype(jnp.int32)
        pltpu.sync_copy(gid_v, gid_hbm)


XCH = 32
NCH = CHUNK // XCH


@functools.partial(
    pl.kernel, mesh=_SC_MESH,
    compiler_params=pltpu.CompilerParams(needs_layout_passes=False),
    out_type=[jax.ShapeDtypeStruct((NPAD, D), jnp.float32),
              jax.ShapeDtypeStruct((NPAD, 128), jnp.float32)],
    scratch_types=[pltpu.VMEM((CHUNK,), jnp.int32)]
                  + [pltpu.VMEM((XCH,), jnp.int32)] * NCH
                  + [pltpu.VMEM((XCH, D), jnp.float32),
                     pltpu.VMEM((XCH, D), jnp.float32),
                     pltpu.VMEM((CHUNK, 128), jnp.float32)]
                  + [pltpu.SemaphoreType.DMA] * 5,
)
def _sc_dispatch(x_hbm, pos_hbm, pk_hbm, xs_hbm, wtws_hbm,
                 pos_v, p0, p1, p2, p3, p4, p5, p6, p7, rows0, rows1,
                 wtws_v, semr0, semr1, semw0, semw1, semw2):
    wid = _wid()
    base = wid * CHUNK
    pcs = (p0, p1, p2, p3, p4, p5, p6, p7)
    rows = (rows0, rows1)
    semr = (semr0, semr1)
    semw = (semw0, semw1)
    pltpu.sync_copy(pos_hbm.at[pl.ds(base, CHUNK)], pos_v)
    pltpu.sync_copy(pk_hbm.at[pl.ds(base, CHUNK)], wtws_v)
    for c in range(NCH):
        pltpu.sync_copy(pos_hbm.at[pl.ds(base + c * XCH, XCH)], pcs[c])
    wtcopy = pltpu.async_copy(wtws_v, wtws_hbm.at[pos_v], semw2)
    rd = [None] * NCH
    wr = [None] * NCH
    rd[0] = pltpu.async_copy(x_hbm.at[pl.ds(base, XCH)], rows[0], semr[0])
    for c in range(NCH):
        if c >= 1:
            wr[c - 1].wait()
        if c + 1 < NCH:
            rd[c + 1] = pltpu.async_copy(
                x_hbm.at[pl.ds(base + (c + 1) * XCH, XCH)],
                rows[(c + 1) % 2], semr[(c + 1) % 2])
        rd[c].wait()
        wr[c] = pltpu.async_copy(rows[c % 2], xs_hbm.at[pcs[c]], semw[c % 2])
    wr[NCH - 1].wait()
    wtcopy.wait()


def _moe_body(gid_ref, xs_ref, wtws_ref,
              sw1_ref, sb1_ref, sw2_ref, sb2_ref, sw3_ref, sb3_ref,
              ew1_ref, eb1_ref, ew2_ref, eb2_ref, ew3_ref, eb3_ref,
              m1w_ref, m1b_ref, m2w_ref, m2b_ref, m3w_ref, m3b_ref,
              out_ref):
    b = pl.program_id(0)

    @pl.when(gid_ref[1, b] == 1)
    def _valid_block():
        _moe_block(xs_ref, wtws_ref,
                   sw1_ref, sb1_ref, sw2_ref, sb2_ref, sw3_ref, sb3_ref,
                   ew1_ref, eb1_ref, ew2_ref, eb2_ref, ew3_ref, eb3_ref,
                   m1w_ref, m1b_ref, m2w_ref, m2b_ref, m3w_ref, m3b_ref,
                   out_ref)


def _moe_block(xs_ref, wtws_ref,
               sw1_ref, sb1_ref, sw2_ref, sb2_ref, sw3_ref, sb3_ref,
               ew1_ref, eb1_ref, ew2_ref, eb2_ref, ew3_ref, eb3_ref,
               m1w_ref, m1b_ref, m2w_ref, m2b_ref, m3w_ref, m3b_ref,
               out_ref):
    xb = xs_ref[...]
    s = jax.nn.silu(_dot(xb, sw1_ref[...]) + sb1_ref[...])
    s = s + _dot(xb, sw3_ref[...]) + sb3_ref[...]
    so = _dot(s, sw2_ref[...]) + sb2_ref[...]
    h = jax.nn.silu(_dot(xb, ew1_ref[0]) + eb1_ref[0])
    h = h + _dot(xb, ew3_ref[0]) + eb3_ref[0]
    eo = _dot(h, ew2_ref[0]) + eb2_ref[0]
    ws = wtws_ref[:, 0:1]
    wt = wtws_ref[:, 1:2]
    c = ws * so + wt * eo + xb
    y = jnp.maximum(_dot(c, m1w_ref[...]) + m1b_ref[...], 0.0)
    y = jnp.maximum(_dot(y, m2w_ref[...]) + m2b_ref[...], 0.0)
    out_ref[...] = jnp.tanh(_dot(y, m3w_ref[...]) + m3b_ref[...])


def _moe(gid, xs, wtws, Sw1, Sb1, Sw2, Sb2, Sw3, Sb3,
         Ew1, Eb1, Ew2, Eb2, Ew3, Eb3, M1w, M1b, M2w, M2b, M3w, M3b):
    full2 = lambda shape: pl.BlockSpec(shape, lambda b, g: (0, 0))
    grid_spec = pltpu.PrefetchScalarGridSpec(
        num_scalar_prefetch=1,
        grid=(NBLK,),
        in_specs=[
            pl.BlockSpec((BT, D), lambda b, g: (b, 0)),
            pl.BlockSpec((BT, 128), lambda b, g: (b, 0)),
            full2((D, H)), full2((1, H)),
            full2((H, D)), full2((1, D)),
            full2((D, H)), full2((1, H)),
            pl.BlockSpec((1, D, H), lambda b, g: (g[0, b], 0, 0)),
            pl.BlockSpec((1, 1, H), lambda b, g: (g[0, b], 0, 0)),
            pl.BlockSpec((1, H, D), lambda b, g: (g[0, b], 0, 0)),
            pl.BlockSpec((1, 1, D), lambda b, g: (g[0, b], 0, 0)),
            pl.BlockSpec((1, D, H), lambda b, g: (g[0, b], 0, 0)),
            pl.BlockSpec((1, 1, H), lambda b, g: (g[0, b], 0, 0)),
            full2((D, 256)), full2((1, 256)),
            full2((256, 256)), full2((1, 256)),
            full2((256, LANES)), full2((1, LANES)),
        ],
        out_specs=pl.BlockSpec((BT, LANES), lambda b, g: (b, 0)),
    )
    m3wp = jnp.pad(M3w, ((0, 0), (0, LANES - 4)))
    m3bp = jnp.pad(M3b.reshape(1, 4), ((0, 0), (0, LANES - 4)))
    return pl.pallas_call(
        _moe_body,
        grid_spec=grid_spec,
        out_shape=jax.ShapeDtypeStruct((NPAD, LANES), jnp.float32),
    )(gid, xs, wtws, Sw1, Sb1.reshape(1, H), Sw2, Sb2.reshape(1, D),
      Sw3, Sb3.reshape(1, H),
      Ew1, Eb1.reshape(NE, 1, H), Ew2, Eb2.reshape(NE, 1, D),
      Ew3, Eb3.reshape(NE, 1, H),
      M1w, M1b.reshape(1, 256), M2w, M2b.reshape(1, 256), m3wp, m3bp)


@functools.partial(
    pl.kernel, mesh=_SC_MESH,
    compiler_params=pltpu.CompilerParams(needs_layout_passes=False,
                                         use_tc_tiling_on_sc=False),
    out_type=jax.ShapeDtypeStruct((B, LANES), jnp.float32),
    scratch_types=[pltpu.VMEM((CHUNK,), jnp.int32),
                   pltpu.VMEM((CHUNK, LANES), jnp.float32),
                   pltpu.SemaphoreType.DMA],
)
def _sc_unperm(head_hbm, pos_hbm, out_hbm, pos_v, rows_v, sem):
    wid = _wid()
    base = wid * CHUNK
    pltpu.sync_copy(pos_hbm.at[pl.ds(base, CHUNK)], pos_v)
    pltpu.async_copy(head_hbm.at[pos_v], rows_v, sem).wait()
    pltpu.sync_copy(rows_v, out_hbm.at[pl.ds(base, CHUNK)])


def kernel(x, Wr1, br1, Wr2, br2, Sw1, Sb1, Sw2, Sb2, Sw3, Sb3,
           Ew1, Eb1, Ew2, Eb2, Ew3, Eb3, M1w, M1b, M2w, M2b, M3w, M3b):
    pk, eidx = _router(x, Wr1, br1, Wr2, br2)
    eidx = eidx.reshape(B)
    hist = _sc_hist(eidx)
    pos, gid = _sc_pos(eidx, hist)
    xs, wtws = _sc_dispatch(x, pos, pk)
    head = _moe(gid, xs, wtws, Sw1, Sb1, Sw2, Sb2, Sw3, Sb3,
                Ew1, Eb1, Ew2, Eb2, Ew3, Eb3, M1w, M1b, M2w, M2b, M3w, M3b)
    out16 = _sc_unperm(head, pos)
    return out16[:, :4]


# speedup vs baseline: 1.1426x; 1.0457x over previous
import functools

import jax
import jax.numpy as jnp
from jax import lax
from jax.experimental import pallas as pl
from jax.experimental.pallas import tpu as pltpu
from jax.experimental.pallas import tpu_sc as plsc

B = 8192
D = 1024
H = 512
E = 8
NE = E - 1
BT = 256
NBLK = B // BT + 8
NPAD = NBLK * BT

NC = 2
NS = 16
NW = NC * NS
CHUNK = B // NW
LANES = 16

_SC_MESH = plsc.VectorSubcoreMesh(core_axis_name="c", subcore_axis_name="s")


def _wid():
    return lax.axis_index("s") * NC + lax.axis_index("c")


def _splat_sum(v):
    incl = plsc.cumsum(v)
    rincl = lax.rev(plsc.cumsum(lax.rev(v, (0,))), (0,))
    return incl + rincl - v


def _dot(a, b):
    return lax.dot_general(a, b, (((1,), (0,)), ((), ())),
                           preferred_element_type=jnp.float32)


RBT = 512


def _router_body(x_ref, wr1_ref, br1_ref, wr2_ref, br2_ref,
                 ws_ref, wt_ref, idx_ref):
    xb = x_ref[...]
    r = _dot(xb, wr1_ref[...]) + br1_ref[...]
    r = 0.5 * r * (1.0 + lax.erf(r * 0.7071067811865476))
    logits = _dot(r, wr2_ref[...]) + br2_ref[...]
    m = jnp.max(logits, axis=1, keepdims=True)
    p = jnp.exp(logits - m)
    w = p / jnp.sum(p, axis=1, keepdims=True)
    shared_w = w[:, 0:1]
    col = lax.broadcasted_iota(jnp.int32, (RBT, E), 1)
    wother = jnp.where(col >= 1, w, -1.0)
    top1 = jnp.max(wother, axis=1, keepdims=True)
    is_max = (wother == top1) & (col >= 1)
    eidx = jnp.min(jnp.where(is_max, col, 127), axis=1, keepdims=True) - 1
    denom = shared_w + top1 + 1e-8
    ws_ref[...] = shared_w / denom
    wt_ref[...] = top1 / denom
    idx_ref[...] = eidx.astype(jnp.int32)


def _router(x, Wr1, br1, Wr2, br2):
    nb = B // RBT
    return pl.pallas_call(
        _router_body,
        grid=(nb,),
        in_specs=[
            pl.BlockSpec((RBT, D), lambda b: (b, 0)),
            pl.BlockSpec((D, H), lambda b: (0, 0)),
            pl.BlockSpec((1, H), lambda b: (0, 0)),
            pl.BlockSpec((H, E), lambda b: (0, 0)),
            pl.BlockSpec((1, E), lambda b: (0, 0)),
        ],
        out_specs=[
            pl.BlockSpec((RBT, 1), lambda b: (b, 0)),
            pl.BlockSpec((RBT, 1), lambda b: (b, 0)),
            pl.BlockSpec((RBT, 1), lambda b: (b, 0)),
        ],
        out_shape=[
            jax.ShapeDtypeStruct((B, 1), jnp.float32),
            jax.ShapeDtypeStruct((B, 1), jnp.float32),
            jax.ShapeDtypeStruct((B, 1), jnp.int32),
        ],
    )(x, Wr1, br1.reshape(1, H), Wr2, br2.reshape(1, E))


@functools.partial(
    pl.kernel, mesh=_SC_MESH,
    compiler_params=pltpu.CompilerParams(needs_layout_passes=False),
    out_type=jax.ShapeDtypeStruct((NW, LANES), jnp.int32),
    scratch_types=[pltpu.VMEM((CHUNK,), jnp.int32),
                   pltpu.VMEM((LANES,), jnp.int32)],
)
def _sc_hist(idx_hbm, hist_hbm, idx_v, hist_v):
    wid = _wid()
    base = wid * CHUNK
    pltpu.sync_copy(idx_hbm.at[pl.ds(base, CHUNK)], idx_v)
    lane = lax.iota(jnp.int32, LANES)
    counts = jnp.zeros((LANES,), jnp.int32)
    for v in range(CHUNK // LANES):
        vec = idx_v[pl.ds(v * LANES, LANES)]
        for e in range(NE):
            ind = (vec == e).astype(jnp.int32)
            counts = counts + jnp.where(lane == e, _splat_sum(ind), 0)
    hist_v[...] = counts
    pltpu.sync_copy(hist_v, hist_hbm.at[wid])


@functools.partial(
    pl.kernel, mesh=_SC_MESH,
    compiler_params=pltpu.CompilerParams(needs_layout_passes=False),
    out_type=[jax.ShapeDtypeStruct((B,), jnp.int32),
              jax.ShapeDtypeStruct((2, 48), jnp.int32)],
    scratch_types=[pltpu.VMEM((CHUNK,), jnp.int32),
                   pltpu.VMEM((NW, LANES), jnp.int32),
                   pltpu.VMEM((LANES,), jnp.int32),
                   pltpu.VMEM((CHUNK,), jnp.int32),
                   pltpu.VMEM((2, 48), jnp.int32)],
)
def _sc_pos(idx_hbm, hist_hbm, pos_hbm, gid_hbm,
            idx_v, hist_v, base_v, pos_v, gid_v):
    wid = _wid()
    base = wid * CHUNK
    lane = lax.iota(jnp.int32, LANES)
    pltpu.sync_copy(hist_hbm, hist_v)
    pltpu.sync_copy(idx_hbm.at[pl.ds(base, CHUNK)], idx_v)

    total = jnp.zeros((LANES,), jnp.int32)
    prefix = jnp.zeros((LANES,), jnp.int32)
    for t in range(NW):
        row = hist_v[t]
        total = total + row
        before = jnp.full((LANES,), t, jnp.int32) < wid
        prefix = prefix + jnp.where(before, row, 0)
    padded = ((total + (BT - 1)) // BT) * BT
    offs = plsc.cumsum(padded) - padded
    base_v[...] = offs + prefix

    counter = jnp.zeros((LANES,), jnp.int32)
    for v in range(CHUNK // LANES):
        vec = idx_v[pl.ds(v * LANES, LANES)]
        rank = jnp.zeros((LANES,), jnp.int32)
        for e in range(NE):
            m = vec == e
            ind = m.astype(jnp.int32)
            excl = plsc.cumsum(ind) - ind
            ce = _splat_sum(jnp.where(lane == e, counter, 0))
            rank = rank + jnp.where(m, excl + ce, 0)
            counter = counter + jnp.where(lane == e, _splat_sum(ind), 0)
        seg = plsc.load_gather(base_v, [vec])
        pos_v[pl.ds(v * LANES, LANES)] = seg + rank
    pltpu.sync_copy(pos_v, pos_hbm.at[pl.ds(base, CHUNK)])

    @pl.when(wid == 0)
    def _gid():
        tp = _splat_sum(padded)
        for v in range(48 // LANES):
            bstart = (lane + v * LANES) * BT
            cnt = jnp.zeros((LANES,), jnp.int32)
            for e in range(NE):
                off_e = _splat_sum(jnp.where(lane == e, offs, 0))
                cnt = cnt + (bstart >= off_e).astype(jnp.int32)
            gid_v[0, pl.ds(v * LANES, LANES)] = jnp.clip(cnt - 1, 0, NE - 1)
            gid_v[1, pl.ds(v * LANES, LANES)] = (bstart < tp).astype(jnp.int32)
        pltpu.sync_copy(gid_v, gid_hbm)


XCH = 32
NCH = CHUNK // XCH


@functools.partial(
    pl.kernel, mesh=_SC_MESH,
    compiler_params=pltpu.CompilerParams(needs_layout_passes=False),
    out_type=[jax.ShapeDtypeStruct((NPAD, D), jnp.float32),
              jax.ShapeDtypeStruct((NPAD, 128), jnp.float32)],
    scratch_types=[pltpu.VMEM((CHUNK,), jnp.int32)]
                  + [pltpu.VMEM((XCH,), jnp.int32)] * NCH
                  + [pltpu.VMEM((XCH, D), jnp.float32),
                     pltpu.VMEM((XCH, D), jnp.float32),
                     pltpu.VMEM((CHUNK, 128), jnp.float32),
                     pltpu.VMEM((CHUNK,), jnp.float32),
                     pltpu.VMEM((CHUNK,), jnp.float32)]
                  + [pltpu.SemaphoreType.DMA] * 5,
)
def _sc_dispatch(x_hbm, pos_hbm, wt_hbm, ws_hbm, xs_hbm, wtws_hbm,
                 pos_v, p0, p1, p2, p3, p4, p5, p6, p7, rows0, rows1,
                 wtws_v, wt_v, ws_v, semr0, semr1, semw0, semw1, semw2):
    wid = _wid()
    base = wid * CHUNK
    lane = lax.iota(jnp.int32, LANES)
    pcs = (p0, p1, p2, p3, p4, p5, p6, p7)
    rows = (rows0, rows1)
    semr = (semr0, semr1)
    semw = (semw0, semw1)
    pltpu.sync_copy(pos_hbm.at[pl.ds(base, CHUNK)], pos_v)
    pltpu.sync_copy(wt_hbm.at[pl.ds(base, CHUNK)], wt_v)
    pltpu.sync_copy(ws_hbm.at[pl.ds(base, CHUNK)], ws_v)
    for c in range(NCH):
        pltpu.sync_copy(pos_hbm.at[pl.ds(base + c * XCH, XCH)], pcs[c])
    zero = jnp.zeros((LANES,), jnp.int32)
    for v in range(CHUNK // LANES):
        row = lane + v * LANES
        plsc.store_scatter(wtws_v, [row, zero], wt_v[pl.ds(v * LANES, LANES)])
        plsc.store_scatter(wtws_v, [row, zero + 1],
                           ws_v[pl.ds(v * LANES, LANES)])
    wtcopy = pltpu.async_copy(wtws_v, wtws_hbm.at[pos_v], semw2)
    rd = [None] * NCH
    wr = [None] * NCH
    rd[0] = pltpu.async_copy(x_hbm.at[pl.ds(base, XCH)], rows[0], semr[0])
    for c in range(NCH):
        if c >= 1:
            wr[c - 1].wait()
        if c + 1 < NCH:
            rd[c + 1] = pltpu.async_copy(
                x_hbm.at[pl.ds(base + (c + 1) * XCH, XCH)],
                rows[(c + 1) % 2], semr[(c + 1) % 2])
        rd[c].wait()
        wr[c] = pltpu.async_copy(rows[c % 2], xs_hbm.at[pcs[c]], semw[c % 2])
    wr[NCH - 1].wait()
    wtcopy.wait()


def _moe_body(gid_ref, xs_ref, wtws_ref,
              sw1_ref, sb1_ref, sw2_ref, sb2_ref, sw3_ref, sb3_ref,
              ew1_ref, eb1_ref, ew2_ref, eb2_ref, ew3_ref, eb3_ref,
              m1w_ref, m1b_ref, m2w_ref, m2b_ref, m3w_ref, m3b_ref,
              out_ref):
    b = pl.program_id(0)

    @pl.when(gid_ref[1, b] == 1)
    def _valid_block():
        _moe_block(xs_ref, wtws_ref,
                   sw1_ref, sb1_ref, sw2_ref, sb2_ref, sw3_ref, sb3_ref,
                   ew1_ref, eb1_ref, ew2_ref, eb2_ref, ew3_ref, eb3_ref,
                   m1w_ref, m1b_ref, m2w_ref, m2b_ref, m3w_ref, m3b_ref,
                   out_ref)


def _moe_block(xs_ref, wtws_ref,
               sw1_ref, sb1_ref, sw2_ref, sb2_ref, sw3_ref, sb3_ref,
               ew1_ref, eb1_ref, ew2_ref, eb2_ref, ew3_ref, eb3_ref,
               m1w_ref, m1b_ref, m2w_ref, m2b_ref, m3w_ref, m3b_ref,
               out_ref):
    xb = xs_ref[...]
    s = jax.nn.silu(_dot(xb, sw1_ref[...]) + sb1_ref[...])
    s = s + _dot(xb, sw3_ref[...]) + sb3_ref[...]
    so = _dot(s, sw2_ref[...]) + sb2_ref[...]
    h = jax.nn.silu(_dot(xb, ew1_ref[0]) + eb1_ref[0])
    h = h + _dot(xb, ew3_ref[0]) + eb3_ref[0]
    eo = _dot(h, ew2_ref[0]) + eb2_ref[0]
    wt = wtws_ref[:, 0:1]
    ws = wtws_ref[:, 1:2]
    c = ws * so + wt * eo + xb
    y = jnp.maximum(_dot(c, m1w_ref[...]) + m1b_ref[...], 0.0)
    y = jnp.maximum(_dot(y, m2w_ref[...]) + m2b_ref[...], 0.0)
    out_ref[...] = jnp.tanh(_dot(y, m3w_ref[...]) + m3b_ref[...])


def _moe(gid, xs, wtws, Sw1, Sb1, Sw2, Sb2, Sw3, Sb3,
         Ew1, Eb1, Ew2, Eb2, Ew3, Eb3, M1w, M1b, M2w, M2b, M3w, M3b):
    full2 = lambda shape: pl.BlockSpec(shape, lambda b, g: (0, 0))
    grid_spec = pltpu.PrefetchScalarGridSpec(
        num_scalar_prefetch=1,
        grid=(NBLK,),
        in_specs=[
            pl.BlockSpec((BT, D), lambda b, g: (b, 0)),
            pl.BlockSpec((BT, 128), lambda b, g: (b, 0)),
            full2((D, H)), full2((1, H)),
            full2((H, D)), full2((1, D)),
            full2((D, H)), full2((1, H)),
            pl.BlockSpec((1, D, H), lambda b, g: (g[0, b], 0, 0)),
            pl.BlockSpec((1, 1, H), lambda b, g: (g[0, b], 0, 0)),
            pl.BlockSpec((1, H, D), lambda b, g: (g[0, b], 0, 0)),
            pl.BlockSpec((1, 1, D), lambda b, g: (g[0, b], 0, 0)),
            pl.BlockSpec((1, D, H), lambda b, g: (g[0, b], 0, 0)),
            pl.BlockSpec((1, 1, H), lambda b, g: (g[0, b], 0, 0)),
            full2((D, 256)), full2((1, 256)),
            full2((256, 256)), full2((1, 256)),
            full2((256, LANES)), full2((1, LANES)),
        ],
        out_specs=pl.BlockSpec((BT, LANES), lambda b, g: (b, 0)),
    )
    m3wp = jnp.pad(M3w, ((0, 0), (0, LANES - 4)))
    m3bp = jnp.pad(M3b.reshape(1, 4), ((0, 0), (0, LANES - 4)))
    return pl.pallas_call(
        _moe_body,
        grid_spec=grid_spec,
        out_shape=jax.ShapeDtypeStruct((NPAD, LANES), jnp.float32),
    )(gid, xs, wtws, Sw1, Sb1.reshape(1, H), Sw2, Sb2.reshape(1, D),
      Sw3, Sb3.reshape(1, H),
      Ew1, Eb1.reshape(NE, 1, H), Ew2, Eb2.reshape(NE, 1, D),
      Ew3, Eb3.reshape(NE, 1, H),
      M1w, M1b.reshape(1, 256), M2w, M2b.reshape(1, 256), m3wp, m3bp)


@functools.partial(
    pl.kernel, mesh=_SC_MESH,
    compiler_params=pltpu.CompilerParams(needs_layout_passes=False,
                                         use_tc_tiling_on_sc=False),
    out_type=jax.ShapeDtypeStruct((B, LANES), jnp.float32),
    scratch_types=[pltpu.VMEM((CHUNK,), jnp.int32),
                   pltpu.VMEM((CHUNK, LANES), jnp.float32),
                   pltpu.SemaphoreType.DMA],
)
def _sc_unperm(head_hbm, pos_hbm, out_hbm, pos_v, rows_v, sem):
    wid = _wid()
    base = wid * CHUNK
    pltpu.sync_copy(pos_hbm.at[pl.ds(base, CHUNK)], pos_v)
    pltpu.async_copy(head_hbm.at[pos_v], rows_v, sem).wait()
    pltpu.sync_copy(rows_v, out_hbm.at[pl.ds(base, CHUNK)])


def kernel(x, Wr1, br1, Wr2, br2, Sw1, Sb1, Sw2, Sb2, Sw3, Sb3,
           Ew1, Eb1, Ew2, Eb2, Ew3, Eb3, M1w, M1b, M2w, M2b, M3w, M3b):
    ws, wt, eidx = _router(x, Wr1, br1, Wr2, br2)
    eidx = eidx.reshape(B)
    hist = _sc_hist(eidx)
    pos, gid = _sc_pos(eidx, hist)
    xs, wtws = _sc_dispatch(x, pos, wt.reshape(B), ws.reshape(B))
    head = _moe(gid, xs, wtws, Sw1, Sb1, Sw2, Sb2, Sw3, Sb3,
                Ew1, Eb1, Ew2, Eb2, Ew3, Eb3, M1w, M1b, M2w, M2b, M3w, M3b)
    out16 = _sc_unperm(head, pos)
    return out16[:, :4]
